# baseline (speedup 1.0000x reference)
import functools

import jax
import jax.numpy as jnp
from jax import lax
from jax.experimental import pallas as pl
from jax.experimental.pallas import tpu as pltpu
from jax.experimental.pallas import tpu_sc as plsc

VOCAB = 100000
EMBED_DIM = 64
BATCH = 4096
HIST = 50

NUM_CORES = 2
NUM_SUBCORES = 16
NUM_WORKERS = NUM_CORES * NUM_SUBCORES

BW = BATCH // NUM_WORKERS
LANES = 16
BGRPS = BW // LANES
EBLKS = EMBED_DIM // 8
OUT_MINOR = 8 * 128
OUT_ROWS = HIST * EMBED_DIM // 8
NBUF = 5


def _sc_embed(table, idx_flat):
    mesh = plsc.VectorSubcoreMesh(core_axis_name="c", subcore_axis_name="s")

    @functools.partial(
        pl.kernel,
        mesh=mesh,
        out_type=jax.ShapeDtypeStruct(
            (OUT_ROWS, NUM_WORKERS, OUT_MINOR), jnp.float32
        ),
        scratch_types=(
            [
                pltpu.VMEM((BW * HIST,), jnp.int32),
                pltpu.VMEM((HIST * BW,), jnp.int32),
            ]
            + [pltpu.VMEM((BW, EMBED_DIM), jnp.float32)] * NBUF
            + [pltpu.VMEM((EBLKS, 1, OUT_MINOR), jnp.float32)] * NBUF
            + [pltpu.SemaphoreType.DMA] * (2 * NBUF)
        ),
        compiler_params=pltpu.CompilerParams(
            use_tc_tiling_on_sc=False, needs_layout_passes=False
        ),
    )
    def k(table_hbm, idx_hbm, out_hbm, idx_v, idxt_v, *bufs_sems):
        wid = lax.axis_index("s") * NUM_CORES + lax.axis_index("c")
        gbufs = bufs_sems[0:NBUF]
        tbufs = bufs_sems[NBUF:2 * NBUF]
        gsems = bufs_sems[2 * NBUF:3 * NBUF]
        osems = bufs_sems[3 * NBUF:4 * NBUF]

        pltpu.sync_copy(
            idx_hbm.at[pl.ds(wid * BW * HIST, BW * HIST)], idx_v
        )

        lane = lax.iota(jnp.int32, LANES)
        rowsel = [lane + bg * LANES for bg in range(BGRPS)]
        stagesel = [(lane + bg * LANES) * HIST for bg in range(BGRPS)]

        @plsc.parallel_loop(0, HIST)
        def _(h):
            for bg in range(BGRPS):
                vals = plsc.load_gather(idx_v, [stagesel[bg] + h])
                idxt_v[pl.ds(h * BW + bg * LANES, LANES)] = vals

        def fire_gather(h, p):
            pltpu.async_copy(
                table_hbm.at[idxt_v.at[pl.ds(h * BW, BW)]], gbufs[p], gsems[p]
            )

        def drain_gather(p):
            pltpu.make_async_copy(
                table_hbm.at[pl.ds(0, BW)], gbufs[p], gsems[p]
            ).wait()

        def fire_out(h, p):
            pltpu.async_copy(
                tbufs[p],
                out_hbm.at[pl.ds(h * EBLKS, EBLKS), pl.ds(wid, 1)],
                osems[p],
            )

        def drain_out(p):
            pltpu.make_async_copy(
                tbufs[p], out_hbm.at[pl.ds(0, EBLKS), pl.ds(0, 1)], osems[p]
            ).wait()

        zerov = jnp.zeros((LANES,), jnp.int32)

        def transpose(p):
            gb, tb = gbufs[p], tbufs[p]

            @plsc.parallel_loop(0, LANES, unroll=2)
            def _(c):
                rot = (c + lane) & 15
                ebrel = rot >> 3
                pivb = (rot & 7) * 128 + lane
                for ek in range(EMBED_DIM // LANES):
                    colv = rot + ek * LANES
                    ebv = ebrel + ek * 2
                    for bg in range(BGRPS):
                        vals = plsc.load_gather(gb, [rowsel[bg], colv])
                        plsc.store_scatter(
                            tb, [ebv, zerov, pivb + bg * LANES], vals
                        )

        for p in range(NBUF):
            fire_gather(p, p)

        def body(g, carry):
            for p in range(NBUF):
                h = g + p

                drain_gather(p)

                @pl.when(h >= NBUF)
                def _():
                    drain_out(p)

                transpose(p)
                fire_out(h, p)

                @pl.when(h + NBUF < HIST)
                def _():
                    fire_gather(h + NBUF, p)
            return carry

        lax.fori_loop(0, HIST // NBUF, lambda i, c: body(i * NBUF, c), 0)
        for p in range(NBUF):
            drain_out(p)

    return k(table, idx_flat)


def kernel(inputs, table):
    idx_flat = inputs.astype(jnp.int32).reshape(BATCH * HIST)
    out3d = _sc_embed(table, idx_flat)
    x5 = out3d.reshape(HIST, EBLKS, NUM_WORKERS, 8, 128)
    return x5.transpose(2, 4, 0, 1, 3).reshape(BATCH, HIST, EMBED_DIM)


# speedup vs baseline: 1.1951x; 1.1951x over previous
import functools

import jax
import jax.numpy as jnp
from jax import lax
from jax.experimental import pallas as pl
from jax.experimental.pallas import tpu as pltpu
from jax.experimental.pallas import tpu_sc as plsc

VOCAB = 100000
EMBED_DIM = 64
BATCH = 4096
HIST = 50

NUM_CORES = 2
NUM_SUBCORES = 16
NUM_WORKERS = NUM_CORES * NUM_SUBCORES

BW = BATCH // NUM_WORKERS
LANES = 16
BGRPS = BW // LANES
EBLKS = EMBED_DIM // 8
OUT_MINOR = 8 * 128
OUT_ROWS = HIST * EMBED_DIM // 8
NBUF = 2


def _sc_embed(table, idx_flat):
    mesh = plsc.VectorSubcoreMesh(core_axis_name="c", subcore_axis_name="s")

    @functools.partial(
        pl.kernel,
        mesh=mesh,
        out_type=jax.ShapeDtypeStruct(
            (OUT_ROWS, NUM_WORKERS, OUT_MINOR), jnp.float32
        ),
        scratch_types=(
            [
                pltpu.VMEM((BW * HIST,), jnp.int32),
                pltpu.VMEM((HIST * BW,), jnp.int32),
            ]
            + [pltpu.VMEM((BW, EMBED_DIM), jnp.float32)] * NBUF
            + [pltpu.VMEM((EBLKS, 1, OUT_MINOR), jnp.float32)] * NBUF
            + [pltpu.SemaphoreType.DMA] * (2 * NBUF)
        ),
        compiler_params=pltpu.CompilerParams(
            use_tc_tiling_on_sc=False, needs_layout_passes=False
        ),
    )
    def k(table_hbm, idx_hbm, out_hbm, idx_v, idxt_v, *bufs_sems):
        wid = lax.axis_index("s") * NUM_CORES + lax.axis_index("c")
        gbufs = bufs_sems[0:NBUF]
        tbufs = bufs_sems[NBUF:2 * NBUF]
        gsems = bufs_sems[2 * NBUF:3 * NBUF]
        osems = bufs_sems[3 * NBUF:4 * NBUF]

        pltpu.sync_copy(
            idx_hbm.at[pl.ds(wid * BW * HIST, BW * HIST)], idx_v
        )

        lane = lax.iota(jnp.int32, LANES)
        rowsel = [lane + bg * LANES for bg in range(BGRPS)]
        stagesel = [(lane + bg * LANES) * HIST for bg in range(BGRPS)]

        @plsc.parallel_loop(0, HIST)
        def _(h):
            for bg in range(BGRPS):
                vals = plsc.load_gather(idx_v, [stagesel[bg] + h])
                idxt_v[pl.ds(h * BW + bg * LANES, LANES)] = vals

        def fire_gather(h, p):
            pltpu.async_copy(
                table_hbm.at[idxt_v.at[pl.ds(h * BW, BW)]], gbufs[p], gsems[p]
            )

        def drain_gather(p):
            pltpu.make_async_copy(
                table_hbm.at[pl.ds(0, BW)], gbufs[p], gsems[p]
            ).wait()

        def fire_out(h, p):
            pltpu.async_copy(
                tbufs[p],
                out_hbm.at[pl.ds(h * EBLKS, EBLKS), pl.ds(wid, 1)],
                osems[p],
            )

        def drain_out(p):
            pltpu.make_async_copy(
                tbufs[p], out_hbm.at[pl.ds(0, EBLKS), pl.ds(0, 1)], osems[p]
            ).wait()

        zerov = jnp.zeros((LANES,), jnp.int32)

        def transpose(p):
            gb, tb = gbufs[p], tbufs[p]

            @plsc.parallel_loop(0, LANES, unroll=2)
            def _(c):
                rot = (c + lane) & 15
                ebrel = rot >> 3
                pivb = (rot & 7) * 128 + lane
                for ek in range(EMBED_DIM // LANES):
                    colv = rot + ek * LANES
                    ebv = ebrel + ek * 2
                    for bg in range(BGRPS):
                        vals = plsc.load_gather(gb, [rowsel[bg], colv])
                        plsc.store_scatter(
                            tb, [ebv, zerov, pivb + bg * LANES], vals
                        )

        fire_gather(0, 0)

        def body(g, carry):
            for p in range(NBUF):
                h = g + p

                @pl.when(h + NBUF - 1 < HIST)
                def _():
                    fire_gather(h + NBUF - 1, (p + NBUF - 1) % NBUF)

                drain_gather(p)

                @pl.when(h >= NBUF)
                def _():
                    drain_out(p)

                transpose(p)
                fire_out(h, p)
            return carry

        lax.fori_loop(0, HIST // NBUF, lambda i, c: body(i * NBUF, c), 0)
        for p in range(NBUF):
            drain_out(p)

    return k(table, idx_flat)


def kernel(inputs, table):
    idx_flat = inputs.astype(jnp.int32).reshape(BATCH * HIST)
    out3d = _sc_embed(table, idx_flat)
    x5 = out3d.reshape(HIST, EBLKS, NUM_WORKERS, 8, 128)
    return x5.transpose(2, 4, 0, 1, 3).reshape(BATCH, HIST, EMBED_DIM)


# speedup vs baseline: 1.2630x; 1.0568x over previous
import functools

import jax
import jax.numpy as jnp
from jax import lax
from jax.experimental import pallas as pl
from jax.experimental.pallas import tpu as pltpu
from jax.experimental.pallas import tpu_sc as plsc

VOCAB = 100000
EMBED_DIM = 64
BATCH = 4096
HIST = 50

NUM_CORES = 2
NUM_SUBCORES = 16
NUM_WORKERS = NUM_CORES * NUM_SUBCORES

BW = BATCH // NUM_WORKERS
LANES = 16
BGRPS = BW // LANES
EBLKS = EMBED_DIM // 8
OUT_MINOR = 8 * 128
OUT_ROWS = HIST * EMBED_DIM // 8
NBUF = 2


def _sc_embed(table, idx_flat):
    mesh = plsc.VectorSubcoreMesh(core_axis_name="c", subcore_axis_name="s")

    @functools.partial(
        pl.kernel,
        mesh=mesh,
        out_type=jax.ShapeDtypeStruct(
            (OUT_ROWS, NUM_WORKERS, OUT_MINOR), jnp.float32
        ),
        scratch_types=(
            [
                pltpu.VMEM((BW * HIST,), jnp.int32),
                pltpu.VMEM((HIST * BW,), jnp.int32),
            ]
            + [pltpu.VMEM((BW, EMBED_DIM), jnp.float32)] * NBUF
            + [pltpu.VMEM((EBLKS, 1, OUT_MINOR), jnp.float32)] * NBUF
            + [pltpu.SemaphoreType.DMA] * (2 * NBUF)
        ),
        compiler_params=pltpu.CompilerParams(
            use_tc_tiling_on_sc=False, needs_layout_passes=False
        ),
    )
    def k(table_hbm, idx_hbm, out_hbm, idx_v, idxt_v, *bufs_sems):
        wid = lax.axis_index("s") * NUM_CORES + lax.axis_index("c")
        gbufs = bufs_sems[0:NBUF]
        tbufs = bufs_sems[NBUF:2 * NBUF]
        gsems = bufs_sems[2 * NBUF:3 * NBUF]
        osems = bufs_sems[3 * NBUF:4 * NBUF]

        pltpu.sync_copy(
            idx_hbm.at[pl.ds(wid * BW * HIST, BW * HIST)], idx_v
        )

        lane = lax.iota(jnp.int32, LANES)
        rowsel = [lane + bg * LANES for bg in range(BGRPS)]
        stagesel = [(lane + bg * LANES) * HIST for bg in range(BGRPS)]

        @plsc.parallel_loop(0, HIST)
        def _(h):
            for bg in range(BGRPS):
                vals = plsc.load_gather(idx_v, [stagesel[bg] + h])
                idxt_v[pl.ds(h * BW + bg * LANES, LANES)] = vals

        def fire_gather(h, p):
            pltpu.async_copy(
                table_hbm.at[idxt_v.at[pl.ds(h * BW, BW)]], gbufs[p], gsems[p]
            )

        def drain_gather(p):
            pltpu.make_async_copy(
                table_hbm.at[pl.ds(0, BW)], gbufs[p], gsems[p]
            ).wait()

        def fire_out(h, p):
            pltpu.async_copy(
                tbufs[p],
                out_hbm.at[pl.ds(h * EBLKS, EBLKS), pl.ds(wid, 1)],
                osems[p],
            )

        def drain_out(p):
            pltpu.make_async_copy(
                tbufs[p], out_hbm.at[pl.ds(0, EBLKS), pl.ds(0, 1)], osems[p]
            ).wait()

        zerov = jnp.zeros((LANES,), jnp.int32)

        def transpose(p):
            gb, tb = gbufs[p], tbufs[p]

            @plsc.parallel_loop(0, LANES * EMBED_DIM // LANES, unroll=2)
            def _(t):
                c = t & 15
                ek = t >> 4
                rot = (c + lane) & 15
                colv = rot + ek * LANES
                ebv = (rot >> 3) + ek * 2
                pivb = (rot & 7) * 128 + lane
                for bg in range(BGRPS):
                    vals = plsc.load_gather(gb, [rowsel[bg], colv])
                    plsc.store_scatter(
                        tb, [ebv, zerov, pivb + bg * LANES], vals
                    )

        for p in range(NBUF - 1):
            fire_gather(p, p)

        def body(g, carry):
            for p in range(NBUF):
                h = g + p

                @pl.when(h + NBUF - 1 < HIST)
                def _():
                    fire_gather(h + NBUF - 1, (p + NBUF - 1) % NBUF)

                drain_gather(p)

                @pl.when(h >= NBUF)
                def _():
                    drain_out(p)

                transpose(p)
                fire_out(h, p)
            return carry

        lax.fori_loop(0, HIST // NBUF, lambda i, c: body(i * NBUF, c), 0)
        for p in range(NBUF):
            drain_out(p)

    return k(table, idx_flat)


def kernel(inputs, table):
    idx_flat = inputs.astype(jnp.int32).reshape(BATCH * HIST)
    out3d = _sc_embed(table, idx_flat)
    x5 = out3d.reshape(HIST, EBLKS, NUM_WORKERS, 8, 128)
    return x5.transpose(2, 4, 0, 1, 3).reshape(BATCH, HIST, EMBED_DIM)


# speedup vs baseline: 1.3704x; 1.0850x over previous
import functools

import jax
import jax.numpy as jnp
from jax import lax
from jax.experimental import pallas as pl
from jax.experimental.pallas import tpu as pltpu
from jax.experimental.pallas import tpu_sc as plsc

VOCAB = 100000
EMBED_DIM = 64
BATCH = 4096
HIST = 50

NUM_CORES = 2
NUM_SUBCORES = 16
NUM_WORKERS = NUM_CORES * NUM_SUBCORES

BW = BATCH // NUM_WORKERS
LANES = 16
BGRPS = BW // LANES
EBLKS = EMBED_DIM // 8
OUT_MINOR = 8 * 128
OUT_ROWS = HIST * EMBED_DIM // 8
NBUF = 5


def _sc_embed(table, idx_flat):
    mesh = plsc.VectorSubcoreMesh(core_axis_name="c", subcore_axis_name="s")

    @functools.partial(
        pl.kernel,
        mesh=mesh,
        out_type=jax.ShapeDtypeStruct(
            (OUT_ROWS, NUM_WORKERS, OUT_MINOR), jnp.float32
        ),
        scratch_types=(
            [
                pltpu.VMEM((BW * HIST,), jnp.int32),
                pltpu.VMEM((HIST * BW,), jnp.int32),
            ]
            + [pltpu.VMEM((BW, EMBED_DIM), jnp.float32)] * NBUF
            + [pltpu.VMEM((EBLKS, 1, OUT_MINOR), jnp.float32)] * NBUF
            + [pltpu.SemaphoreType.DMA] * (2 * NBUF)
        ),
        compiler_params=pltpu.CompilerParams(
            use_tc_tiling_on_sc=False, needs_layout_passes=False
        ),
    )
    def k(table_hbm, idx_hbm, out_hbm, idx_v, idxt_v, *bufs_sems):
        wid = lax.axis_index("s") * NUM_CORES + lax.axis_index("c")
        gbufs = bufs_sems[0:NBUF]
        tbufs = bufs_sems[NBUF:2 * NBUF]
        gsems = bufs_sems[2 * NBUF:3 * NBUF]
        osems = bufs_sems[3 * NBUF:4 * NBUF]

        pltpu.sync_copy(
            idx_hbm.at[pl.ds(wid * BW * HIST, BW * HIST)], idx_v
        )

        lane = lax.iota(jnp.int32, LANES)
        rowsel = [lane + bg * LANES for bg in range(BGRPS)]
        stagesel = [(lane + bg * LANES) * HIST for bg in range(BGRPS)]

        @plsc.parallel_loop(0, HIST)
        def _(h):
            for bg in range(BGRPS):
                vals = plsc.load_gather(idx_v, [stagesel[bg] + h])
                idxt_v[pl.ds(h * BW + bg * LANES, LANES)] = vals

        def fire_gather(h, p):
            pltpu.async_copy(
                table_hbm.at[idxt_v.at[pl.ds(h * BW, BW)]], gbufs[p], gsems[p]
            )

        def drain_gather(p):
            pltpu.make_async_copy(
                table_hbm.at[pl.ds(0, BW)], gbufs[p], gsems[p]
            ).wait()

        def fire_out(h, p):
            pltpu.async_copy(
                tbufs[p],
                out_hbm.at[pl.ds(h * EBLKS, EBLKS), pl.ds(wid, 1)],
                osems[p],
            )

        def drain_out(p):
            pltpu.make_async_copy(
                tbufs[p], out_hbm.at[pl.ds(0, EBLKS), pl.ds(0, 1)], osems[p]
            ).wait()

        zerov = jnp.zeros((LANES,), jnp.int32)

        def transpose(p):
            gb, tb = gbufs[p], tbufs[p]

            @plsc.parallel_loop(0, LANES * EMBED_DIM // LANES, unroll=2)
            def _(t):
                c = t & 15
                ek = t >> 4
                rot = (c + lane) & 15
                colv = rot + ek * LANES
                ebv = (rot >> 3) + ek * 2
                pivb = (rot & 7) * 128 + lane
                for bg in range(BGRPS):
                    vals = plsc.load_gather(gb, [rowsel[bg], colv])
                    plsc.store_scatter(
                        tb, [ebv, zerov, pivb + bg * LANES], vals
                    )

        for p in range(NBUF - 1):
            fire_gather(p, p)

        def body(g, carry):
            for p in range(NBUF):
                h = g + p

                @pl.when(h + NBUF - 1 < HIST)
                def _():
                    fire_gather(h + NBUF - 1, (p + NBUF - 1) % NBUF)

                drain_gather(p)

                @pl.when(h >= NBUF)
                def _():
                    drain_out(p)

                transpose(p)
                fire_out(h, p)
            return carry

        lax.fori_loop(0, HIST // NBUF, lambda i, c: body(i * NBUF, c), 0)
        for p in range(NBUF):
            drain_out(p)

    return k(table, idx_flat)


def kernel(inputs, table):
    idx_flat = inputs.astype(jnp.int32).reshape(BATCH * HIST)
    out3d = _sc_embed(table, idx_flat)
    x5 = out3d.reshape(HIST, EBLKS, NUM_WORKERS, 8, 128)
    return x5.transpose(2, 4, 0, 1, 3).reshape(BATCH, HIST, EMBED_DIM)
